# baseline (speedup 1.0000x reference)
import functools

import jax
import jax.numpy as jnp
from jax import lax
from jax.experimental import pallas as pl
from jax.experimental.pallas import tpu as pltpu
from jax.experimental.pallas import tpu_sc as plsc


@functools.lru_cache(maxsize=None)
def _make_gather(V, D, Bt, H):
    info = plsc.get_sparse_core_info()
    NC, NS = info.num_cores, info.num_subcores
    NW = NC * NS
    assert Bt % NW == 0
    bt_per_w = Bt // NW
    b_per_w = bt_per_w * H
    NBUF = 2
    CHB = 8
    CH = CHB * H
    assert bt_per_w % (NBUF * CHB) == 0
    n_groups = bt_per_w // (NBUF * CHB)
    mesh = plsc.VectorSubcoreMesh(core_axis_name="c", subcore_axis_name="s")

    @functools.partial(
        pl.kernel,
        mesh=mesh,
        out_type=jax.ShapeDtypeStruct((Bt, H, D), jnp.float32),
        compiler_params=pltpu.CompilerParams(use_tc_tiling_on_sc=False),
        scratch_types=[
            pltpu.VMEM((b_per_w,), jnp.int32),
            pltpu.VMEM((NBUF, CH, D), jnp.float32),
        ]
        + [pltpu.SemaphoreType.DMA] * (2 * NBUF),
    )
    def k(idx_hbm, table_hbm, out_hbm, idx_v, rows_v, *sems):
        sem_g = sems[:NBUF]
        sem_w = sems[NBUF:]
        wid = lax.axis_index("s") * NC + lax.axis_index("c")
        base = wid * b_per_w
        bt_base = wid * bt_per_w
        pltpu.sync_copy(idx_hbm.at[pl.ds(base, b_per_w)], idx_v)

        def gather(i, n):
            pltpu.async_copy(
                table_hbm.at[idx_v.at[pl.ds(i * CH, CH)]], rows_v.at[n], sem_g[n]
            )

        def wait_gather(n):
            pltpu.make_async_copy(
                table_hbm.at[pl.ds(0, CH)], rows_v.at[n], sem_g[n]
            ).wait()

        def writeback(i, n):
            for j in range(CHB):
                pltpu.async_copy(
                    rows_v.at[n].at[pl.ds(j * H, H)],
                    out_hbm.at[bt_base + i * CHB + j],
                    sem_w[n],
                )

        def wait_writeback(n):
            for j in range(CHB):
                pltpu.make_async_copy(
                    rows_v.at[n].at[pl.ds(j * H, H)],
                    out_hbm.at[0],
                    sem_w[n],
                ).wait()

        for n in range(NBUF):
            gather(n, n)

        def group_body(g, carry):
            i0 = g * NBUF
            for n in range(NBUF):
                wait_gather(n)
                writeback(i0 + n, n)
            for n in range(NBUF):
                wait_writeback(n)
                gather(i0 + NBUF + n, n)
            return carry

        lax.fori_loop(0, n_groups - 1, group_body, 0)

        i0 = (n_groups - 1) * NBUF
        for n in range(NBUF):
            wait_gather(n)
            writeback(i0 + n, n)
        for n in range(NBUF):
            wait_writeback(n)

    return k


def kernel(token_ids, embed):
    Bt, H = token_ids.shape
    V, D = embed.shape
    flat = token_ids.reshape(-1).astype(jnp.int32)
    return _make_gather(V, D, Bt, H)(flat, embed)


# speedup vs baseline: 1.0032x; 1.0032x over previous
import functools

import jax
import jax.numpy as jnp
from jax import lax
from jax.experimental import pallas as pl
from jax.experimental.pallas import tpu as pltpu
from jax.experimental.pallas import tpu_sc as plsc


@functools.lru_cache(maxsize=None)
def _make_gather(V, D, B):
    info = plsc.get_sparse_core_info()
    NC, NS = info.num_cores, info.num_subcores
    NW = NC * NS
    assert B % NW == 0
    b_per_w = B // NW
    NBUF = 4
    CH = 320
    assert b_per_w % (NBUF * CH) == 0
    n_groups = b_per_w // (NBUF * CH)
    mesh = plsc.VectorSubcoreMesh(core_axis_name="c", subcore_axis_name="s")

    @functools.partial(
        pl.kernel,
        mesh=mesh,
        out_type=jax.ShapeDtypeStruct((B, D), jnp.float32),
        compiler_params=pltpu.CompilerParams(use_tc_tiling_on_sc=False),
        scratch_types=[
            pltpu.VMEM((b_per_w,), jnp.int32),
            pltpu.VMEM((NBUF, CH, D), jnp.float32),
        ]
        + [pltpu.SemaphoreType.DMA] * (2 * NBUF),
    )
    def k(idx_hbm, table_hbm, out_hbm, idx_v, rows_v, *sems):
        sem_g = sems[:NBUF]
        sem_w = sems[NBUF:]
        wid = lax.axis_index("s") * NC + lax.axis_index("c")
        base = wid * b_per_w
        pltpu.sync_copy(idx_hbm.at[pl.ds(base, b_per_w)], idx_v)

        def gather(i, b):
            pltpu.async_copy(
                table_hbm.at[idx_v.at[pl.ds(i * CH, CH)]], rows_v.at[b], sem_g[b]
            )

        def wait_gather(b):
            pltpu.make_async_copy(
                out_hbm.at[pl.ds(0, CH)], rows_v.at[b], sem_g[b]
            ).wait()

        def writeback(i, b):
            pltpu.async_copy(
                rows_v.at[b], out_hbm.at[pl.ds(base + i * CH, CH)], sem_w[b]
            )

        def wait_writeback(b):
            pltpu.make_async_copy(
                rows_v.at[b], out_hbm.at[pl.ds(0, CH)], sem_w[b]
            ).wait()

        for b in range(NBUF):
            gather(b, b)

        def group_body(g, carry):
            i0 = g * NBUF
            for b in range(NBUF):
                wait_gather(b)
                writeback(i0 + b, b)
            for b in range(NBUF):
                wait_writeback(b)
                gather(i0 + NBUF + b, b)
            return carry

        lax.fori_loop(0, n_groups - 1, group_body, 0)

        i0 = (n_groups - 1) * NBUF
        for b in range(NBUF):
            wait_gather(b)
            writeback(i0 + b, b)
        for b in range(NBUF):
            wait_writeback(b)

    return k


def kernel(token_ids, embed):
    Bt, H = token_ids.shape
    V, D = embed.shape
    flat = token_ids.reshape(-1).astype(jnp.int32)
    out = _make_gather(V, D, flat.shape[0])(flat, embed)
    return out.reshape(Bt, H, D)
